# baseline (speedup 1.0000x reference)
import dataclasses
import functools

import jax
import jax.numpy as jnp
from jax import lax
from jax.experimental import pallas as pl
from jax.experimental.pallas import tpu as pltpu
from jax.experimental.pallas import tpu_sc as plsc

N = 10000
E = 320000
D = 128
G = 16
NTILES = 32
CHUNK = 128
NBUF = 1
CPT0 = 100
CPT1 = 60
CPT = max(CPT0, CPT1)
EPAD = 16 * (CPT0 + CPT1) * CHUNK
NPAD = 10112
ROWS_PER_TILE = NPAD // 16
STRIPES = (128, 128, 128, 128, 120)
VECS = CHUNK // 16

BLK = 400
NBLK = N // BLK


def _sc_compiler_params():
    cp = pltpu.CompilerParams()
    if "needs_layout_passes" in pltpu.CompilerParams.__dataclass_fields__:
        cp = dataclasses.replace(cp, needs_layout_passes=False)
    return cp


def _sc_aggregate(h, edges):
    mesh = plsc.VectorSubcoreMesh(core_axis_name="c", subcore_axis_name="s")

    @functools.partial(
        pl.kernel,
        out_type=jax.ShapeDtypeStruct((2, NPAD, D), jnp.float32),
        mesh=mesh,
        scratch_types=(
            [pltpu.VMEM((CPT, CHUNK), jnp.int32)]
            + [pltpu.VMEM((CHUNK, D), jnp.float32) for _ in range(NBUF)]
            + [pltpu.VMEM((NBUF, CHUNK), jnp.int32)] * 2
            + [pltpu.VMEM_SHARED((NPAD, D), jnp.float32)]
            + [pltpu.SemaphoreType.DMA for _ in range(2 * NBUF + 1)]
        ),
        compiler_params=_sc_compiler_params(),
    )
    def agg_kernel(h_hbm, e_hbm, out_hbm, slab_v, *rest):
        rows = rest[:NBUF]
        us_v = rest[NBUF]
        ud_v = rest[NBUF + 1]
        acc = rest[NBUF + 2]
        sem_g = rest[NBUF + 3:2 * NBUF + 3]
        sem_s = rest[2 * NBUF + 3:3 * NBUF + 3]
        sem_f = rest[3 * NBUF + 3]
        c = lax.axis_index("c")
        s = lax.axis_index("s")
        wid = c * 16 + s
        base = s * ROWS_PER_TILE

        fetch = pltpu.async_copy(e_hbm.at[wid], slab_v, sem_f)

        def unpack(j, sl):
            for k in range(VECS):
                p = slab_v[j, pl.ds(k * 16, 16)]
                us_v[sl, pl.ds(k * 16, 16)] = lax.shift_right_logical(p, 14)
                ud_v[sl, pl.ds(k * 16, 16)] = lax.bitwise_and(p, 16383)

        def gather(b):
            return pltpu.make_async_copy(
                h_hbm.at[us_v.at[b]], rows[b], sem_g[b])

        def scatter(b):
            return pltpu.make_async_copy(
                rows[b], acc.at[ud_v.at[b]], sem_s[b])

        @pl.loop(0, CHUNK)
        def _(i):
            @pl.loop(0, D, step=16)
            def _(j):
                rows[0][i, pl.ds(j, 16)] = jnp.zeros((16,), jnp.float32)

        def stripe_zero(off, sz):
            return pltpu.make_async_copy(
                rows[0].at[pl.ds(0, sz)],
                acc.at[pl.ds(base + off, sz)], sem_s[0])

        off = 0
        for sz in STRIPES:
            stripe_zero(off, sz).start()
            off += sz
        off = 0
        for sz in STRIPES:
            stripe_zero(off, sz).wait()
            off += sz

        n_c = CPT0 - c * (CPT0 - CPT1)

        fetch.wait()
        plsc.subcore_barrier()

        @pl.loop(0, n_c)
        def _(j):
            unpack(j, 0)
            g = gather(0)
            g.start()
            g.wait()
            d = scatter(0)
            d.start(add=True)
            d.wait()

        plsc.subcore_barrier()

        def stripe_out(off, sz):
            r0 = base + off
            return pltpu.make_async_copy(
                acc.at[pl.ds(r0, sz)], out_hbm.at[c].at[pl.ds(r0, sz)],
                sem_g[0])

        off = 0
        for sz in STRIPES:
            stripe_out(off, sz).start()
            off += sz
        off = 0
        for sz in STRIPES:
            stripe_out(off, sz).wait()
            off += sz

    return agg_kernel(h, edges)


def _sc_degree(edges):
    mesh = plsc.VectorSubcoreMesh(core_axis_name="c", subcore_axis_name="s")

    @functools.partial(
        pl.kernel,
        out_type=jax.ShapeDtypeStruct((NTILES * NPAD,), jnp.float32),
        mesh=mesh,
        scratch_types=[
            pltpu.VMEM((CPT, CHUNK), jnp.int32),
            pltpu.VMEM((NPAD,), jnp.float32),
            pltpu.SemaphoreType.DMA,
        ],
        compiler_params=_sc_compiler_params(),
    )
    def deg_kernel(e_hbm, cnt_hbm, slab_v, cnt_v, sem):
        c = lax.axis_index("c")
        s = lax.axis_index("s")
        wid = c * 16 + s

        fetch = pltpu.async_copy(e_hbm.at[wid], slab_v, sem)

        @pl.loop(0, NPAD, step=16)
        def _(i):
            cnt_v[pl.ds(i, 16)] = jnp.zeros((16,), jnp.float32)

        fetch.wait()

        @pl.loop(0, CPT)
        def _(j):
            for k in range(VECS):
                ids = lax.bitwise_and(slab_v[j, pl.ds(k * 16, 16)], 16383)
                run, last = plsc.scan_count(ids)
                plsc.addupdate_scatter(
                    cnt_v, [ids], run.astype(jnp.float32), mask=last)

        pltpu.sync_copy(cnt_v, cnt_hbm.at[pl.ds(wid * NPAD, NPAD)])

    return deg_kernel(edges)


def _tc_embed(x, w0t, b0r):
    def body(x_ref, w_ref, b_ref, o_ref):
        y = jnp.dot(x_ref[...], w_ref[...], preferred_element_type=jnp.float32)
        o_ref[...] = jnp.maximum(y + b_ref[...], 0.0)

    return pl.pallas_call(
        body,
        grid=(NBLK,),
        in_specs=[
            pl.BlockSpec((BLK, D), lambda i: (i, 0)),
            pl.BlockSpec((D, D), lambda i: (0, 0)),
            pl.BlockSpec((1, D), lambda i: (0, 0)),
        ],
        out_specs=pl.BlockSpec((BLK, D), lambda i: (i, 0)),
        out_shape=jax.ShapeDtypeStruct((N, D), jnp.float32),
    )(x, w0t, b0r)


def _tc_conv(acc, cnt2d, h, wlt, wrt, blr):
    def body(a_ref, c_ref, h_ref, wl_ref, wr_ref, b_ref, o_ref):
        cnt = jnp.sum(c_ref[:, 0, 0, :], axis=0).reshape(BLK, 1)
        agg = (a_ref[0] + a_ref[1]) / jnp.maximum(cnt, 1.0)
        y = jnp.dot(agg, wl_ref[...], preferred_element_type=jnp.float32)
        y = y + jnp.dot(h_ref[...], wr_ref[...], preferred_element_type=jnp.float32)
        o_ref[...] = jnp.maximum(y + b_ref[...], 0.0)

    return pl.pallas_call(
        body,
        grid=(NBLK,),
        in_specs=[
            pl.BlockSpec((2, BLK, D), lambda i: (0, i, 0)),
            pl.BlockSpec((NTILES, 1, 1, BLK), lambda i: (0, i, 0, 0)),
            pl.BlockSpec((BLK, D), lambda i: (i, 0)),
            pl.BlockSpec((D, D), lambda i: (0, 0)),
            pl.BlockSpec((D, D), lambda i: (0, 0)),
            pl.BlockSpec((1, D), lambda i: (0, 0)),
        ],
        out_specs=pl.BlockSpec((BLK, D), lambda i: (i, 0)),
        out_shape=jax.ShapeDtypeStruct((N, D), jnp.float32),
    )(acc, cnt2d, h, wlt, wrt, blr)


def _tc_pool(h, batch3d, w1t, b1r):
    def body(h_ref, b_ref, w_ref, bias_ref, o_ref, ps, pc):
        i = pl.program_id(0)

        @pl.when(i == 0)
        def _():
            ps[...] = jnp.zeros((G, D), jnp.float32)
            pc[...] = jnp.zeros((G, 1), jnp.float32)

        ids = b_ref[0, 0]
        gids = lax.broadcasted_iota(jnp.int32, (G, BLK), 0)
        mask = (gids == ids[None, :]).astype(jnp.float32)
        ps[...] += jnp.dot(mask, h_ref[...], preferred_element_type=jnp.float32)
        pc[...] += jnp.sum(mask, axis=1, keepdims=True)

        @pl.when(i == NBLK - 1)
        def _():
            pooled = ps[...] / jnp.maximum(pc[...], 1.0)
            o_ref[...] = jnp.dot(pooled, w_ref[...],
                                 preferred_element_type=jnp.float32) + bias_ref[...]

    return pl.pallas_call(
        body,
        grid=(NBLK,),
        in_specs=[
            pl.BlockSpec((BLK, D), lambda i: (i, 0)),
            pl.BlockSpec((1, 1, BLK), lambda i: (i, 0, 0)),
            pl.BlockSpec((D, D), lambda i: (0, 0)),
            pl.BlockSpec((1, D), lambda i: (0, 0)),
        ],
        out_specs=pl.BlockSpec((G, D), lambda i: (0, 0)),
        out_shape=jax.ShapeDtypeStruct((G, D), jnp.float32),
        scratch_shapes=[
            pltpu.VMEM((G, D), jnp.float32),
            pltpu.VMEM((G, 1), jnp.float32),
        ],
    )(h, batch3d, w1t, b1r)


def kernel(x, edge_index, edge_attr, batch,
           lin0_w, lin0_b,
           conv1_wl, conv1_bl, conv1_wr,
           conv2_wl, conv2_bl, conv2_wr,
           lin1_w, lin1_b):
    pad = EPAD - E
    srcs = jnp.concatenate([edge_index[0], jnp.zeros((pad,), jnp.int32)])
    dsts = jnp.concatenate([edge_index[1], jnp.full((pad,), N, jnp.int32)])
    packed = (srcs << 14) | dsts
    n0 = 16 * CPT0 * CHUNK
    e0 = packed[:n0].reshape(16, CPT0, CHUNK)
    e1 = packed[n0:].reshape(16, CPT1, CHUNK)
    if CPT0 < CPT:
        e0 = jnp.concatenate(
            [e0, jnp.full((16, CPT - CPT0, CHUNK), N, jnp.int32)], axis=1)
    if CPT1 < CPT:
        e1 = jnp.concatenate(
            [e1, jnp.full((16, CPT - CPT1, CHUNK), N, jnp.int32)], axis=1)
    edges = jnp.concatenate([e0, e1], axis=0)

    cnt = _sc_degree(edges)
    cnt2d = cnt.reshape(NTILES, NPAD)[:, :N].reshape(NTILES, NBLK, 1, BLK)
    h0 = _tc_embed(x, lin0_w.T, lin0_b.reshape(1, D))
    acc1 = _sc_aggregate(h0, edges)
    h1 = _tc_conv(acc1, cnt2d, h0, conv1_wl.T, conv1_wr.T, conv1_bl.reshape(1, D))
    acc2 = _sc_aggregate(h1, edges)
    h2 = _tc_conv(acc2, cnt2d, h1, conv2_wl.T, conv2_wr.T, conv2_bl.reshape(1, D))

    batch3d = batch.reshape(NBLK, 1, BLK)
    return _tc_pool(h2, batch3d, lin1_w.T, lin1_b.reshape(1, D))


# speedup vs baseline: 1.5137x; 1.5137x over previous
import dataclasses
import functools

import jax
import jax.numpy as jnp
from jax import lax
from jax.experimental import pallas as pl
from jax.experimental.pallas import tpu as pltpu
from jax.experimental.pallas import tpu_sc as plsc

N = 10000
E = 320000
D = 128
G = 16
NTILES = 32
CHUNK = 128
CPT = 79
EPAD = NTILES * CPT * CHUNK
NPAD = 10240
ROWS_PER_TILE = NPAD // 16
ZCH = ROWS_PER_TILE // CHUNK
VECS = CHUNK // 16

BLK = 400
NBLK = N // BLK


def _sc_aggregate(h, edges, with_cnt):
    mesh = plsc.VectorSubcoreMesh(core_axis_name="c", subcore_axis_name="s")

    out_type = [jax.ShapeDtypeStruct((2, NPAD, D), jnp.float32)]
    scratch = [
        pltpu.VMEM((CPT, 2, CHUNK), jnp.int32),
        pltpu.VMEM((CHUNK, D), jnp.float32),
        pltpu.VMEM_SHARED((NPAD, D), jnp.float32),
        pltpu.SemaphoreType.DMA,
    ]
    if with_cnt:
        out_type.append(jax.ShapeDtypeStruct((NTILES * NPAD,), jnp.float32))
        scratch.append(pltpu.VMEM((NPAD,), jnp.float32))

    cp = pltpu.CompilerParams()
    if "needs_layout_passes" in pltpu.CompilerParams.__dataclass_fields__:
        cp = dataclasses.replace(cp, needs_layout_passes=False)

    @functools.partial(
        pl.kernel, out_type=out_type, mesh=mesh, scratch_types=scratch,
        compiler_params=cp,
    )
    def agg_kernel(h_hbm, e_hbm, out_hbm, *rest):
        if with_cnt:
            cnt_hbm = rest[0]
            rest = rest[1:]
        idx_v = rest[0]
        rows_v = rest[1]
        acc = rest[2]
        sem = rest[3]
        if with_cnt:
            cnt_v = rest[4]
        c = lax.axis_index("c")
        s = lax.axis_index("s")
        wid = c * 16 + s
        base = s * ROWS_PER_TILE

        fetch = pltpu.async_copy(e_hbm.at[wid], idx_v, sem)

        @pl.loop(0, CHUNK)
        def _(i):
            @pl.loop(0, D, step=16)
            def _(j):
                rows_v[i, pl.ds(j, 16)] = jnp.zeros((16,), jnp.float32)

        if with_cnt:
            @pl.loop(0, NPAD, step=16)
            def _(i):
                cnt_v[pl.ds(i, 16)] = jnp.zeros((16,), jnp.float32)

        for z in range(ZCH):
            pltpu.sync_copy(rows_v, acc.at[pl.ds(base + z * CHUNK, CHUNK)])
        fetch.wait()
        plsc.subcore_barrier()

        @pl.loop(0, CPT)
        def _(j):
            gather = pltpu.async_copy(h_hbm.at[idx_v.at[j, 0]], rows_v, sem)
            if with_cnt:
                for k in range(VECS):
                    ids = idx_v[j, 1, pl.ds(k * 16, 16)]
                    run, last = plsc.scan_count(ids)
                    plsc.addupdate_scatter(
                        cnt_v, [ids], run.astype(jnp.float32), mask=last)
            gather.wait()
            pltpu.sync_copy(rows_v, acc.at[idx_v.at[j, 1]], add=True)

        plsc.subcore_barrier()
        for z in range(ZCH):
            r0 = base + z * CHUNK
            pltpu.sync_copy(acc.at[pl.ds(r0, CHUNK)], out_hbm.at[c].at[pl.ds(r0, CHUNK)])
        if with_cnt:
            pltpu.sync_copy(cnt_v, cnt_hbm.at[pl.ds(wid * NPAD, NPAD)])

    return agg_kernel(h, edges)


def _tc_embed(x, w0t, b0r):
    def body(x_ref, w_ref, b_ref, o_ref):
        y = jnp.dot(x_ref[...], w_ref[...], preferred_element_type=jnp.float32)
        o_ref[...] = jnp.maximum(y + b_ref[...], 0.0)

    return pl.pallas_call(
        body,
        grid=(NBLK,),
        in_specs=[
            pl.BlockSpec((BLK, D), lambda i: (i, 0)),
            pl.BlockSpec((D, D), lambda i: (0, 0)),
            pl.BlockSpec((1, D), lambda i: (0, 0)),
        ],
        out_specs=pl.BlockSpec((BLK, D), lambda i: (i, 0)),
        out_shape=jax.ShapeDtypeStruct((N, D), jnp.float32),
    )(x, w0t, b0r)


def _tc_conv(acc, cnt2d, h, wlt, wrt, blr):
    def body(a_ref, c_ref, h_ref, wl_ref, wr_ref, b_ref, o_ref):
        cnt = jnp.sum(c_ref[:, 0, 0, :], axis=0).reshape(BLK, 1)
        agg = (a_ref[0] + a_ref[1]) / jnp.maximum(cnt, 1.0)
        y = jnp.dot(agg, wl_ref[...], preferred_element_type=jnp.float32)
        y = y + jnp.dot(h_ref[...], wr_ref[...], preferred_element_type=jnp.float32)
        o_ref[...] = jnp.maximum(y + b_ref[...], 0.0)

    return pl.pallas_call(
        body,
        grid=(NBLK,),
        in_specs=[
            pl.BlockSpec((2, BLK, D), lambda i: (0, i, 0)),
            pl.BlockSpec((NTILES, 1, 1, BLK), lambda i: (0, i, 0, 0)),
            pl.BlockSpec((BLK, D), lambda i: (i, 0)),
            pl.BlockSpec((D, D), lambda i: (0, 0)),
            pl.BlockSpec((D, D), lambda i: (0, 0)),
            pl.BlockSpec((1, D), lambda i: (0, 0)),
        ],
        out_specs=pl.BlockSpec((BLK, D), lambda i: (i, 0)),
        out_shape=jax.ShapeDtypeStruct((N, D), jnp.float32),
    )(acc, cnt2d, h, wlt, wrt, blr)


def _tc_pool(h, batch3d, w1t, b1r):
    def body(h_ref, b_ref, w_ref, bias_ref, o_ref, ps, pc):
        i = pl.program_id(0)

        @pl.when(i == 0)
        def _():
            ps[...] = jnp.zeros((G, D), jnp.float32)
            pc[...] = jnp.zeros((G, 1), jnp.float32)

        ids = b_ref[0, 0]
        gids = lax.broadcasted_iota(jnp.int32, (G, BLK), 0)
        mask = (gids == ids[None, :]).astype(jnp.float32)
        ps[...] += jnp.dot(mask, h_ref[...], preferred_element_type=jnp.float32)
        pc[...] += jnp.sum(mask, axis=1, keepdims=True)

        @pl.when(i == NBLK - 1)
        def _():
            pooled = ps[...] / jnp.maximum(pc[...], 1.0)
            o_ref[...] = jnp.dot(pooled, w_ref[...],
                                 preferred_element_type=jnp.float32) + bias_ref[...]

    return pl.pallas_call(
        body,
        grid=(NBLK,),
        in_specs=[
            pl.BlockSpec((BLK, D), lambda i: (i, 0)),
            pl.BlockSpec((1, 1, BLK), lambda i: (i, 0, 0)),
            pl.BlockSpec((D, D), lambda i: (0, 0)),
            pl.BlockSpec((1, D), lambda i: (0, 0)),
        ],
        out_specs=pl.BlockSpec((G, D), lambda i: (0, 0)),
        out_shape=jax.ShapeDtypeStruct((G, D), jnp.float32),
        scratch_shapes=[
            pltpu.VMEM((G, D), jnp.float32),
            pltpu.VMEM((G, 1), jnp.float32),
        ],
    )(h, batch3d, w1t, b1r)


def kernel(x, edge_index, edge_attr, batch,
           lin0_w, lin0_b,
           conv1_wl, conv1_bl, conv1_wr,
           conv2_wl, conv2_bl, conv2_wr,
           lin1_w, lin1_b):
    pad = EPAD - E
    srcs = jnp.concatenate([edge_index[0], jnp.zeros((pad,), jnp.int32)])
    dsts = jnp.concatenate([edge_index[1], jnp.full((pad,), N, jnp.int32)])
    edges = jnp.stack([srcs.reshape(NTILES, CPT, CHUNK),
                       dsts.reshape(NTILES, CPT, CHUNK)], axis=2)

    h0 = _tc_embed(x, lin0_w.T, lin0_b.reshape(1, D))
    acc1, cnt = _sc_aggregate(h0, edges, with_cnt=True)
    cnt2d = cnt.reshape(NTILES, NPAD)[:, :N].reshape(NTILES, NBLK, 1, BLK)
    h1 = _tc_conv(acc1, cnt2d, h0, conv1_wl.T, conv1_wr.T, conv1_bl.reshape(1, D))
    (acc2,) = _sc_aggregate(h1, edges, with_cnt=False)
    h2 = _tc_conv(acc2, cnt2d, h1, conv2_wl.T, conv2_wr.T, conv2_bl.reshape(1, D))

    batch3d = batch.reshape(NBLK, 1, BLK)
    return _tc_pool(h2, batch3d, lin1_w.T, lin1_b.reshape(1, D))
